# baseline (speedup 1.0000x reference)
import functools

import jax
import jax.numpy as jnp
from jax import lax
from jax.experimental import pallas as pl
from jax.experimental.pallas import tpu as pltpu
from jax.experimental.pallas import tpu_sc as plsc

N_PROMPT = 170
SEQ = 512
TOK = SEQ - N_PROMPT
BSZ = 1024
D = 128
NC, NS = 2, 16
NW = NC * NS
B_PER_W = BSZ // NW
IDX_CHUNKS = 3
PAD_TOK = IDX_CHUNKS * D


def _body(idx_hbm, table_hbm, pna, p1, p2, p3, p4, p5, psep, out_hbm,
          idx_v, rows_v, prompt_v, sem):
    c = lax.axis_index("c")
    s = lax.axis_index("s")
    wid = s * NC + c

    pltpu.sync_copy(pna, prompt_v.at[pl.ds(0, 1)])
    pltpu.sync_copy(p1, prompt_v.at[pl.ds(1, 34)])
    pltpu.sync_copy(p2, prompt_v.at[pl.ds(35, 34)])
    pltpu.sync_copy(p3, prompt_v.at[pl.ds(69, 34)])
    pltpu.sync_copy(p4, prompt_v.at[pl.ds(103, 33)])
    pltpu.sync_copy(p5, prompt_v.at[pl.ds(136, 33)])
    pltpu.sync_copy(psep, prompt_v.at[pl.ds(169, 1)])

    def step(i, carry):
        b = wid * B_PER_W + i
        pltpu.sync_copy(idx_hbm.at[b], idx_v)
        for j in range(IDX_CHUNKS):
            pltpu.async_copy(
                table_hbm.at[idx_v.at[j]],
                rows_v.at[pl.ds(j * D, D)],
                sem,
            ).wait()
        base = b * SEQ
        pltpu.sync_copy(rows_v.at[pl.ds(0, TOK)],
                        out_hbm.at[pl.ds(base, TOK)])
        pltpu.sync_copy(prompt_v, out_hbm.at[pl.ds(base + TOK, N_PROMPT)])
        return carry

    lax.fori_loop(0, B_PER_W, step, 0)


_sc_call = pl.kernel(
    _body,
    out_type=jax.ShapeDtypeStruct((BSZ * SEQ, D), jnp.float32),
    mesh=plsc.VectorSubcoreMesh(
        core_axis_name="c", subcore_axis_name="s",
        num_cores=NC, num_subcores=NS,
    ),
    scratch_types=[
        pltpu.VMEM((IDX_CHUNKS, D), jnp.int32),
        pltpu.VMEM((PAD_TOK, D), jnp.float32),
        pltpu.VMEM((N_PROMPT, D), jnp.float32),
        pltpu.SemaphoreType.DMA,
    ],
    compiler_params=pltpu.CompilerParams(use_tc_tiling_on_sc=False),
)


@jax.jit
def kernel(tokens, embed_table, prompt_na, prompt1, prompt2, prompt3,
           prompt4, prompt5, prompt_sep):
    idx = jnp.pad(tokens[:, :TOK], ((0, 0), (0, PAD_TOK - TOK)))
    idx = idx.reshape(BSZ, IDX_CHUNKS, D)
    out = _sc_call(idx, embed_table, prompt_na, prompt1, prompt2, prompt3,
                   prompt4, prompt5, prompt_sep)
    return out.reshape(BSZ, SEQ, D)


# speedup vs baseline: 1.8266x; 1.8266x over previous
import jax
import jax.numpy as jnp
from jax import lax
from jax.experimental import pallas as pl
from jax.experimental.pallas import tpu as pltpu
from jax.experimental.pallas import tpu_sc as plsc

N_PROMPT = 170
SEQ = 512
TOK = SEQ - N_PROMPT
BSZ = 1024
D = 128
NC, NS = 2, 16
NW = NC * NS
B_PER_W = BSZ // NW
CH = 120
NCHUNK = 3
TAIL = TOK - 2 * CH
RING = 6
ROUNDS = B_PER_W * NCHUNK // RING
CH_PER_W = B_PER_W * NCHUNK


def _body(idx_hbm, table_hbm, pna, p1, p2, p3, p4, p5, psep, out_hbm,
          idx_v, prompt_v, bufs, gsems, wsems, psem):
    c = lax.axis_index("c")
    s = lax.axis_index("s")
    wid = s * NC + c
    b0 = wid * B_PER_W

    pltpu.sync_copy(pna, prompt_v.at[pl.ds(0, 1)])
    pltpu.sync_copy(p1, prompt_v.at[pl.ds(1, 34)])
    pltpu.sync_copy(p2, prompt_v.at[pl.ds(35, 34)])
    pltpu.sync_copy(p3, prompt_v.at[pl.ds(69, 34)])
    pltpu.sync_copy(p4, prompt_v.at[pl.ds(103, 33)])
    pltpu.sync_copy(p5, prompt_v.at[pl.ds(136, 33)])
    pltpu.sync_copy(psep, prompt_v.at[pl.ds(169, 1)])

    pltpu.sync_copy(idx_hbm.at[pl.ds(wid * CH_PER_W, CH_PER_W)], idx_v)

    def fire_gather(chunk, r):
        pltpu.async_copy(table_hbm.at[idx_v.at[chunk]], bufs[r], gsems[r])

    def round_body(t, prefetch):
        for r in range(RING):
            chunk = t * RING + r
            j = r % NCHUNK
            b = b0 + 2 * t + r // NCHUNK
            pltpu.make_async_copy(table_hbm.at[idx_v.at[chunk]],
                                  bufs[r], gsems[r]).wait()
            n = CH if j < NCHUNK - 1 else TAIL
            dst = out_hbm.at[pl.ds(b * SEQ + j * CH, n)]
            wr = pltpu.async_copy(bufs[r].at[pl.ds(0, n)], dst, wsems[r])
            if j == NCHUNK - 1:
                pltpu.async_copy(
                    prompt_v, out_hbm.at[pl.ds(b * SEQ + TOK, N_PROMPT)],
                    psem)
            wr.wait()
            if prefetch:
                fire_gather(chunk + RING, r)
        for k in range(2):
            b = b0 + 2 * t + k
            pltpu.make_async_copy(
                prompt_v, out_hbm.at[pl.ds(b * SEQ + TOK, N_PROMPT)],
                psem).wait()
        return 0

    for r in range(RING):
        fire_gather(r, r)
    lax.fori_loop(0, ROUNDS - 1, lambda t, u: round_body(t, True), 0)
    round_body(ROUNDS - 1, False)


_sc_call = pl.kernel(
    _body,
    out_type=jax.ShapeDtypeStruct((BSZ * SEQ, D), jnp.float32),
    mesh=plsc.VectorSubcoreMesh(
        core_axis_name="c", subcore_axis_name="s",
        num_cores=NC, num_subcores=NS,
    ),
    scratch_types=[
        pltpu.VMEM((CH_PER_W, CH), jnp.int32),
        pltpu.VMEM((N_PROMPT, D), jnp.float32),
        [pltpu.VMEM((CH, D), jnp.float32)] * RING,
        [pltpu.SemaphoreType.DMA] * RING,
        [pltpu.SemaphoreType.DMA] * RING,
        pltpu.SemaphoreType.DMA,
    ],
    compiler_params=pltpu.CompilerParams(use_tc_tiling_on_sc=False),
)


@jax.jit
def kernel(tokens, embed_table, prompt_na, prompt1, prompt2, prompt3,
           prompt4, prompt5, prompt_sep):
    idx = jnp.pad(tokens[:, :TOK], ((0, 0), (0, NCHUNK * CH - TOK)))
    idx = idx.reshape(BSZ * NCHUNK, CH)
    out = _sc_call(idx, embed_table, prompt_na, prompt1, prompt2, prompt3,
                   prompt4, prompt5, prompt_sep)
    return out.reshape(BSZ, SEQ, D)


# speedup vs baseline: 1.8326x; 1.0033x over previous
import jax
import jax.numpy as jnp
from jax import lax
from jax.experimental import pallas as pl
from jax.experimental.pallas import tpu as pltpu
from jax.experimental.pallas import tpu_sc as plsc

N_PROMPT = 170
SEQ = 512
TOK = SEQ - N_PROMPT
BSZ = 1024
D = 128
NC, NS = 2, 16
NW = NC * NS
B_PER_W = BSZ // NW
CH = 120
NCHUNK = 3
TAIL = TOK - 2 * CH
RING = 6
ROUNDS = B_PER_W * NCHUNK // RING
CH_PER_W = B_PER_W * NCHUNK


def _body(idx_hbm, table_hbm, pna, p1, p2, p3, p4, p5, psep, out_hbm,
          idx_v, prompt_v, bufs, gsems, wsems, psem):
    c = lax.axis_index("c")
    s = lax.axis_index("s")
    wid = s * NC + c
    b0 = wid * B_PER_W

    pltpu.sync_copy(pna, prompt_v.at[pl.ds(0, 1)])
    pltpu.sync_copy(p1, prompt_v.at[pl.ds(1, 34)])
    pltpu.sync_copy(p2, prompt_v.at[pl.ds(35, 34)])
    pltpu.sync_copy(p3, prompt_v.at[pl.ds(69, 34)])
    pltpu.sync_copy(p4, prompt_v.at[pl.ds(103, 33)])
    pltpu.sync_copy(p5, prompt_v.at[pl.ds(136, 33)])
    pltpu.sync_copy(psep, prompt_v.at[pl.ds(169, 1)])

    pltpu.sync_copy(idx_hbm.at[pl.ds(wid * CH_PER_W, CH_PER_W)], idx_v)

    def fire_gather(chunk, r):
        pltpu.async_copy(table_hbm.at[idx_v.at[chunk]], bufs[r], gsems[r])

    def wdesc(t, r):
        j = r % NCHUNK
        b = b0 + 2 * t + r // NCHUNK
        n = CH if j < NCHUNK - 1 else TAIL
        return pltpu.make_async_copy(
            bufs[r].at[pl.ds(0, n)],
            out_hbm.at[pl.ds(b * SEQ + j * CH, n)], wsems[r])

    PD = RING - 2

    def slot(t, r, waitw, prefetch):
        chunk = t * RING + r
        pltpu.make_async_copy(table_hbm.at[idx_v.at[chunk]],
                              bufs[r], gsems[r]).wait()
        wdesc(t, r).start()
        if r % NCHUNK == NCHUNK - 1:
            b = b0 + 2 * t + r // NCHUNK
            pltpu.async_copy(
                prompt_v, out_hbm.at[pl.ds(b * SEQ + TOK, N_PROMPT)], psem)
        if waitw:
            if r >= 2:
                wdesc(t, r - 2).wait()
            else:
                wdesc(t - 1, r + 4).wait()
        if prefetch:
            fire_gather(chunk + PD, (r + PD) % RING)

    for r in range(PD):
        fire_gather(r, r)
    for r in range(RING):
        slot(0, r, r >= 2, True)
    lax.fori_loop(
        1, ROUNDS - 1,
        lambda t, u: [slot(t, r, True, True) for r in range(RING)] and u, 0)
    for r in range(RING):
        slot(ROUNDS - 1, r, True, r < 2)
    wdesc(ROUNDS - 1, RING - 2).wait()
    wdesc(ROUNDS - 1, RING - 1).wait()

    def drain_prompt(i, u):
        pltpu.make_async_copy(
            prompt_v,
            out_hbm.at[pl.ds((b0 + i) * SEQ + TOK, N_PROMPT)], psem).wait()
        return u

    lax.fori_loop(0, B_PER_W, drain_prompt, 0)


_sc_call = pl.kernel(
    _body,
    out_type=jax.ShapeDtypeStruct((BSZ * SEQ, D), jnp.float32),
    mesh=plsc.VectorSubcoreMesh(
        core_axis_name="c", subcore_axis_name="s",
        num_cores=NC, num_subcores=NS,
    ),
    scratch_types=[
        pltpu.VMEM((CH_PER_W, CH), jnp.int32),
        pltpu.VMEM((N_PROMPT, D), jnp.float32),
        [pltpu.VMEM((CH, D), jnp.float32)] * RING,
        [pltpu.SemaphoreType.DMA] * RING,
        [pltpu.SemaphoreType.DMA] * RING,
        pltpu.SemaphoreType.DMA,
    ],
    compiler_params=pltpu.CompilerParams(use_tc_tiling_on_sc=False),
)


@jax.jit
def kernel(tokens, embed_table, prompt_na, prompt1, prompt2, prompt3,
           prompt4, prompt5, prompt_sep):
    idx = jnp.pad(tokens[:, :TOK], ((0, 0), (0, NCHUNK * CH - TOK)))
    idx = idx.reshape(BSZ * NCHUNK, CH)
    out = _sc_call(idx, embed_table, prompt_na, prompt1, prompt2, prompt3,
                   prompt4, prompt5, prompt_sep)
    return out.reshape(BSZ, SEQ, D)


# speedup vs baseline: 6.5204x; 3.5580x over previous
import jax
import jax.numpy as jnp
from jax import lax
from jax.experimental import pallas as pl
from jax.experimental.pallas import tpu as pltpu
from jax.experimental.pallas import tpu_sc as plsc

N_PROMPT = 170
SEQ = 512
TOK = SEQ - N_PROMPT
BSZ = 1024
D = 128
NC, NS = 2, 16
NW = NC * NS
B_PER_W = BSZ // NW
IDXROW = 344


def _body(idx_hbm, table_hbm, pna, p1, p2, p3, p4, p5, psep, out_hbm,
          idx_v, prompt_v, bufs, gsems, wsems):
    c = lax.axis_index("c")
    s = lax.axis_index("s")
    wid = s * NC + c
    b0 = wid * B_PER_W

    pltpu.sync_copy(pna, prompt_v.at[pl.ds(0, 1)])
    pltpu.sync_copy(p1, prompt_v.at[pl.ds(1, 34)])
    pltpu.sync_copy(p2, prompt_v.at[pl.ds(35, 34)])
    pltpu.sync_copy(p3, prompt_v.at[pl.ds(69, 34)])
    pltpu.sync_copy(p4, prompt_v.at[pl.ds(103, 33)])
    pltpu.sync_copy(p5, prompt_v.at[pl.ds(136, 33)])
    pltpu.sync_copy(psep, prompt_v.at[pl.ds(169, 1)])

    pltpu.sync_copy(idx_hbm.at[pl.ds(b0, B_PER_W)], idx_v)

    def fire_gather(i, r):
        pltpu.async_copy(table_hbm.at[idx_v.at[i]], bufs[r], gsems[r])

    def gwait(i, r):
        pltpu.make_async_copy(table_hbm.at[idx_v.at[i]],
                              bufs[r], gsems[r]).wait()

    def fire_writes(i, r):
        b = b0 + i
        pltpu.async_copy(bufs[r].at[pl.ds(0, TOK)],
                         out_hbm.at[pl.ds(b * SEQ, TOK)], wsems[r])
        pltpu.async_copy(prompt_v,
                         out_hbm.at[pl.ds(b * SEQ + TOK, N_PROMPT)],
                         wsems[r])

    def wait_writes(i, r):
        b = b0 + i
        pltpu.make_async_copy(bufs[r].at[pl.ds(0, TOK)],
                              out_hbm.at[pl.ds(b * SEQ, TOK)],
                              wsems[r]).wait()
        pltpu.make_async_copy(prompt_v,
                              out_hbm.at[pl.ds(b * SEQ + TOK, N_PROMPT)],
                              wsems[r]).wait()

    fire_gather(0, 0)
    fire_gather(1, 1)

    def round_body(t, last):
        for r in range(2):
            i = 2 * t + r
            gwait(i, r)
            fire_writes(i, r)
            if not last:
                wait_writes(i, r)
                fire_gather(i + 2, r)
        return 0

    lax.fori_loop(0, B_PER_W // 2 - 1,
                  lambda t, u: round_body(t, False), 0)
    round_body(B_PER_W // 2 - 1, True)
    wait_writes(B_PER_W - 2, 0)
    wait_writes(B_PER_W - 1, 1)


_sc_call = pl.kernel(
    _body,
    out_type=jax.ShapeDtypeStruct((BSZ * SEQ, D), jnp.float32),
    mesh=plsc.VectorSubcoreMesh(
        core_axis_name="c", subcore_axis_name="s",
        num_cores=NC, num_subcores=NS,
    ),
    scratch_types=[
        pltpu.VMEM((B_PER_W, IDXROW), jnp.int32),
        pltpu.VMEM((N_PROMPT, D), jnp.float32),
        [pltpu.VMEM((IDXROW, D), jnp.float32)] * 2,
        [pltpu.SemaphoreType.DMA] * 2,
        [pltpu.SemaphoreType.DMA] * 2,
    ],
    compiler_params=pltpu.CompilerParams(use_tc_tiling_on_sc=False),
)


@jax.jit
def kernel(tokens, embed_table, prompt_na, prompt1, prompt2, prompt3,
           prompt4, prompt5, prompt_sep):
    idx = jnp.pad(tokens[:, :TOK], ((0, 0), (0, IDXROW - TOK)))
    out = _sc_call(idx, embed_table, prompt_na, prompt1, prompt2, prompt3,
                   prompt4, prompt5, prompt_sep)
    return out.reshape(BSZ, SEQ, D)
